# baseline (speedup 1.0000x reference)
import functools

import jax
import jax.numpy as jnp
from jax import lax
from jax.experimental import pallas as pl
from jax.experimental.pallas import tpu as pltpu
from jax.experimental.pallas import tpu_sc as plsc

_D = 256
_L = 16
_NPROTO = 2


def _permute(x, idxv):
    dn = lax.GatherDimensionNumbers(
        offset_dims=(), collapsed_slice_dims=(0,), start_index_map=(0,))
    return lax.gather(x, idxv[:, None], dn, slice_sizes=(1,),
                      mode=lax.GatherScatterMode.PROMISE_IN_BOUNDS)


def _sc_body(z_hbm, par_hbm, loss_hbm, label_hbm, u_hbm,
             z_v, proto_v, par_v, loss_v, label_v, u_v):
    cid = lax.axis_index("c")
    sid = lax.axis_index("s")
    wid = sid * 2 + cid

    @pl.when(wid == 0)
    def _():
        pltpu.sync_copy(z_hbm, z_v)
        pltpu.sync_copy(par_hbm, par_v)
        pltpu.sync_copy(z_hbm, proto_v.at[pl.ds(0, _D)])
        pltpu.sync_copy(z_hbm, proto_v.at[pl.ds(_D, _D)])

        idx = lax.iota(jnp.int32, _L)
        zero_i = jnp.zeros((_L,), jnp.int32)
        pv = par_v[...]
        bv = _permute(pv, zero_i)
        gv = _permute(pv, zero_i + 1)

        dvecs = []
        for r in range(_NPROTO):
            acc = jnp.zeros((_L,), jnp.float32)
            for i in range(_D // _L):
                prow = proto_v[pl.ds(r * _D + i * _L, _L)]
                zv = z_v[pl.ds(i * _L, _L)]
                diff = prow - zv
                acc = acc + diff * diff
            for sh in (8, 4, 2, 1):
                acc = acc + _permute(acc, (idx + sh) % _L)
            dvecs.append(acc)
        d0, d1 = dvecs

        u_v[...] = 1.0 / (1.0 + jnp.exp(-((d0 - bv) / gv)))

        l0 = -d0
        l1 = -d1
        label_v[...] = jnp.where(l0 >= l1, zero_i, zero_i + 1)

        mx = jnp.maximum(l0, l1)
        s = jnp.exp(l0 - mx) + jnp.exp(l1 - mx)
        y = jnp.full((_L,), 0.6931472)
        for _ in range(4):
            y = y + s * jnp.exp(-y) - 1.0
        loss_v[...] = y

        pltpu.sync_copy(loss_v, loss_hbm)
        pltpu.sync_copy(label_v, label_hbm)
        pltpu.sync_copy(u_v, u_hbm)


@jax.jit
def _run(zf, par):
    mesh = plsc.VectorSubcoreMesh(core_axis_name="c", subcore_axis_name="s")
    f = pl.kernel(
        _sc_body,
        out_type=(
            jax.ShapeDtypeStruct((_L,), jnp.float32),
            jax.ShapeDtypeStruct((_L,), jnp.int32),
            jax.ShapeDtypeStruct((_L,), jnp.float32),
        ),
        mesh=mesh,
        scratch_types=[
            pltpu.VMEM((_D,), jnp.float32),
            pltpu.VMEM((_NPROTO * _D,), jnp.float32),
            pltpu.VMEM((_L,), jnp.float32),
            pltpu.VMEM((_L,), jnp.float32),
            pltpu.VMEM((_L,), jnp.int32),
            pltpu.VMEM((_L,), jnp.float32),
        ],
        name="prototype_memory_sc",
    )
    return f(zf, par)


def kernel(z, beta, gamma):
    zf = z.reshape(_D)
    par = jnp.concatenate(
        [beta, gamma, jnp.zeros((_L - 2,), jnp.float32)])
    loss16, label16, u16 = _run(zf, par)
    return (loss16[0], label16[0:1], u16[0:1])
